# baseline (speedup 1.0000x reference)
import jax
import jax.numpy as jnp
from jax import lax
from jax.experimental import pallas as pl

E = 16
TOP_K = 2
CONF_THRESH = 0.7
TILE = 1024


def _route(x, w):
    logits = jnp.dot(x, w, preferred_element_type=jnp.float32)
    lt = logits.T
    m = jnp.max(lt, axis=0, keepdims=True)
    e = jnp.exp(lt - m)
    z = jnp.sum(e, axis=0, keepdims=True)
    p = e / z

    eidx = lax.broadcasted_iota(jnp.int32, p.shape, 0)
    big = jnp.full(p.shape, E, jnp.int32)

    m1 = jnp.max(p, axis=0, keepdims=True)
    i1 = jnp.min(jnp.where(p == m1, eidx, big), axis=0, keepdims=True)
    p2 = jnp.where(eidx == i1, -1.0, p)
    m2 = jnp.max(p2, axis=0, keepdims=True)
    hit2 = jnp.logical_and(p == m2, eidx != i1)
    i2 = jnp.min(jnp.where(hit2, eidx, big), axis=0, keepdims=True)

    keep = m1 >= CONF_THRESH
    i1 = jnp.where(keep, i1, -1)
    i2 = jnp.where(keep, i2, -1)

    wts = jnp.concatenate([m1, m2], axis=0)
    idx = jnp.concatenate([i1, i2], axis=0)
    return wts, idx


SPLIT = 2


def _router_body(xa_ref, xb_ref, w_ref, wts_ref, idx_ref):
    w = w_ref[...]
    wa, ia = _route(xa_ref[...], w)
    wb, ib = _route(xb_ref[...], w)
    wts_ref[...] = jnp.concatenate([wa, wb], axis=1)
    idx_ref[...] = jnp.concatenate([ia, ib], axis=1)


def kernel(x, W_g):
    B, S, D = x.shape
    N = B * S
    x2 = x.reshape(N, D)
    grid = (N // (TILE * SPLIT),)
    wts_t, idx_t = pl.pallas_call(
        _router_body,
        grid=grid,
        in_specs=[
            pl.BlockSpec((TILE, D), lambda i: (SPLIT * i, 0)),
            pl.BlockSpec((TILE, D), lambda i: (SPLIT * i + 1, 0)),
            pl.BlockSpec((D, E), lambda i: (0, 0)),
        ],
        out_specs=[
            pl.BlockSpec((TOP_K, TILE * SPLIT), lambda i: (0, i)),
            pl.BlockSpec((TOP_K, TILE * SPLIT), lambda i: (0, i)),
        ],
        out_shape=[
            jax.ShapeDtypeStruct((TOP_K, N), jnp.float32),
            jax.ShapeDtypeStruct((TOP_K, N), jnp.int32),
        ],
    )(x2, x2, W_g)
    wts = wts_t.T.reshape(B, S, TOP_K)
    idx = idx_t.T.reshape(B, S, TOP_K)
    return wts, idx


# speedup vs baseline: 1.0528x; 1.0528x over previous
import jax
import jax.numpy as jnp
from jax import lax
from jax.experimental import pallas as pl

E = 16
TOP_K = 2
CONF_THRESH = 0.7
TILE = 512


def _route(x, w):
    logits = jnp.dot(x, w, preferred_element_type=jnp.float32)
    lt = logits.T
    m = jnp.max(lt, axis=0, keepdims=True)
    e = jnp.exp(lt - m)
    z = jnp.sum(e, axis=0, keepdims=True)
    p = e / z

    eidx = lax.broadcasted_iota(jnp.int32, p.shape, 0)
    big = jnp.full(p.shape, E, jnp.int32)

    m1 = jnp.max(p, axis=0, keepdims=True)
    i1 = jnp.min(jnp.where(p == m1, eidx, big), axis=0, keepdims=True)
    p2 = jnp.where(eidx == i1, -1.0, p)
    m2 = jnp.max(p2, axis=0, keepdims=True)
    hit2 = jnp.logical_and(p == m2, eidx != i1)
    i2 = jnp.min(jnp.where(hit2, eidx, big), axis=0, keepdims=True)

    keep = m1 >= CONF_THRESH
    i1 = jnp.where(keep, i1, -1)
    i2 = jnp.where(keep, i2, -1)

    wts = jnp.concatenate([m1, m2], axis=0)
    idx = jnp.concatenate([i1, i2], axis=0)
    return wts, idx


SPLIT = 2


def _router_body(xa_ref, xb_ref, w_ref, wts_ref, idx_ref):
    w = w_ref[...]
    wa, ia = _route(xa_ref[...], w)
    wb, ib = _route(xb_ref[...], w)
    wts_ref[...] = jnp.concatenate([wa, wb], axis=1)
    idx_ref[...] = jnp.concatenate([ia, ib], axis=1)


def kernel(x, W_g):
    B, S, D = x.shape
    N = B * S
    x2 = x.reshape(N, D)
    grid = (N // (TILE * SPLIT),)
    wts_t, idx_t = pl.pallas_call(
        _router_body,
        grid=grid,
        in_specs=[
            pl.BlockSpec((TILE, D), lambda i: (SPLIT * i, 0)),
            pl.BlockSpec((TILE, D), lambda i: (SPLIT * i + 1, 0)),
            pl.BlockSpec((D, E), lambda i: (0, 0)),
        ],
        out_specs=[
            pl.BlockSpec((TOP_K, TILE * SPLIT), lambda i: (0, i)),
            pl.BlockSpec((TOP_K, TILE * SPLIT), lambda i: (0, i)),
        ],
        out_shape=[
            jax.ShapeDtypeStruct((TOP_K, N), jnp.float32),
            jax.ShapeDtypeStruct((TOP_K, N), jnp.int32),
        ],
    )(x2, x2, W_g)
    wts = wts_t.T.reshape(B, S, TOP_K)
    idx = idx_t.T.reshape(B, S, TOP_K)
    return wts, idx


# speedup vs baseline: 1.0910x; 1.0363x over previous
import jax
import jax.numpy as jnp
from jax import lax
from jax.experimental import pallas as pl

E = 16
TOP_K = 2
CONF_THRESH = 0.7
TILE = 1024


def _router_body(x_ref, w_ref, wts_ref, idx_ref):
    logits = jnp.dot(x_ref[...], w_ref[...], preferred_element_type=jnp.float32)
    lt = logits.T
    m = jnp.max(lt, axis=0, keepdims=True)
    e = jnp.exp(lt - m)
    z = jnp.sum(e, axis=0, keepdims=True)
    p = e / z

    eidx = lax.broadcasted_iota(jnp.int32, p.shape, 0)
    big = jnp.full(p.shape, E, jnp.int32)

    m1 = jnp.max(p, axis=0, keepdims=True)
    i1 = jnp.min(jnp.where(p == m1, eidx, big), axis=0, keepdims=True)
    p2 = jnp.where(eidx == i1, -1.0, p)
    m2 = jnp.max(p2, axis=0, keepdims=True)
    hit2 = jnp.logical_and(p == m2, eidx != i1)
    i2 = jnp.min(jnp.where(hit2, eidx, big), axis=0, keepdims=True)

    keep = m1 >= CONF_THRESH
    i1 = jnp.where(keep, i1, -1)
    i2 = jnp.where(keep, i2, -1)

    wts_ref[...] = jnp.concatenate([m1, m2], axis=0)
    idx_ref[...] = jnp.concatenate([i1, i2], axis=0)


def kernel(x, W_g):
    B, S, D = x.shape
    N = B * S
    x2 = x.reshape(N, D)
    grid = (N // TILE,)
    wts_t, idx_t = pl.pallas_call(
        _router_body,
        grid=grid,
        in_specs=[
            pl.BlockSpec((TILE, D), lambda i: (i, 0)),
            pl.BlockSpec((D, E), lambda i: (0, 0)),
        ],
        out_specs=[
            pl.BlockSpec((TOP_K, TILE), lambda i: (0, i)),
            pl.BlockSpec((TOP_K, TILE), lambda i: (0, i)),
        ],
        out_shape=[
            jax.ShapeDtypeStruct((TOP_K, N), jnp.float32),
            jax.ShapeDtypeStruct((TOP_K, N), jnp.int32),
        ],
    )(x2, W_g)
    wts = wts_t.T.reshape(B, S, TOP_K)
    idx = idx_t.T.reshape(B, S, TOP_K)
    return wts, idx
